# baseline (speedup 1.0000x reference)
import functools

import jax
import jax.numpy as jnp
from jax import lax
from jax.experimental import pallas as pl
from jax.experimental.pallas import tpu as pltpu
from jax.experimental.pallas import tpu_sc as plsc

_D = 128
_NC = 2
_NS = 16
_NW = _NC * _NS
_CHUNK = 64
_NBUF = 9
_LOOKAHEAD = 4


@functools.lru_cache(maxsize=None)
def _make_gather(n_tokens):
    per_w = n_tokens // _NW
    n_chunk = per_w // _CHUNK
    assert per_w * _NW == n_tokens and n_chunk * _CHUNK == per_w
    assert n_chunk % _NBUF == 1

    mesh = plsc.VectorSubcoreMesh(core_axis_name="c", subcore_axis_name="s")

    @functools.partial(
        pl.kernel,
        out_type=jax.ShapeDtypeStruct((n_tokens, _D), jnp.float32),
        mesh=mesh,
        scratch_types=[
            pltpu.VMEM((n_chunk, _CHUNK), jnp.int32),
            pltpu.VMEM((_NBUF, _CHUNK, _D), jnp.float32),
            [pltpu.SemaphoreType.DMA] * _NBUF,
            [pltpu.SemaphoreType.DMA] * _NBUF,
        ],
    )
    def gather_kernel(idx_hbm, table_hbm, out_hbm, idx_v, rows_v, gsems, osems):
        wid = lax.axis_index("s") * _NC + lax.axis_index("c")
        base = wid * per_w
        pltpu.sync_copy(idx_hbm.at[wid], idx_v)

        def start_gather(j, b):
            pltpu.async_copy(table_hbm.at[idx_v.at[j]], rows_v.at[b], gsems[b])

        def wait_gather(b):
            pltpu.make_async_copy(
                table_hbm.at[idx_v.at[0]], rows_v.at[b], gsems[b]
            ).wait()

        def start_out(j, b):
            pltpu.async_copy(
                rows_v.at[b],
                out_hbm.at[pl.ds(base + j * _CHUNK, _CHUNK)],
                osems[b],
            )

        def wait_out(b):
            pltpu.make_async_copy(
                rows_v.at[b], out_hbm.at[pl.ds(base, _CHUNK)], osems[b]
            ).wait()

        for j in range(_LOOKAHEAD):
            start_gather(j, j)

        @pl.loop(0, n_chunk - 1, step=_NBUF)
        def _group(g):
            for b in range(_NBUF):
                j = g + b
                wait_gather(b)
                start_out(j, b)
                s = (b + _LOOKAHEAD) % _NBUF

                @pl.when(j + _LOOKAHEAD < n_chunk)
                def _issue():
                    @pl.when(j >= _NBUF - _LOOKAHEAD)
                    def _recycle():
                        wait_out(s)

                    start_gather(j + _LOOKAHEAD, s)

        tail = n_chunk - 1
        b_tail = tail % _NBUF
        wait_gather(b_tail)
        start_out(tail, b_tail)
        for b in range(_NBUF):
            wait_out(b)

    return gather_kernel


def kernel(token_ids, table):
    b, l = token_ids.shape
    n = b * l
    idx = token_ids.reshape(_NW, n // (_NW * _CHUNK), _CHUNK).astype(jnp.int32)
    out = _make_gather(n)(idx, table)
    return out.reshape(b, l, _D)


# speedup vs baseline: 1.0141x; 1.0141x over previous
import functools

import jax
import jax.numpy as jnp
from jax import lax
from jax.experimental import pallas as pl
from jax.experimental.pallas import tpu as pltpu
from jax.experimental.pallas import tpu_sc as plsc

_D = 128
_NC = 2
_NS = 16
_NW = _NC * _NS
_CHUNK = 128
_NBUF = 7
_LOOKAHEAD = 5


@functools.lru_cache(maxsize=None)
def _make_gather(n_tokens):
    per_w = n_tokens // _NW
    n_chunk = per_w // _CHUNK
    assert per_w * _NW == n_tokens and n_chunk * _CHUNK == per_w
    assert n_chunk % _NBUF == 1

    mesh = plsc.VectorSubcoreMesh(core_axis_name="c", subcore_axis_name="s")

    @functools.partial(
        pl.kernel,
        out_type=jax.ShapeDtypeStruct((n_tokens, _D), jnp.float32),
        mesh=mesh,
        scratch_types=[
            pltpu.VMEM((n_chunk, _CHUNK), jnp.int32),
            pltpu.VMEM((_NBUF, _CHUNK, _D), jnp.float32),
            [pltpu.SemaphoreType.DMA] * _NBUF,
            [pltpu.SemaphoreType.DMA] * _NBUF,
        ],
    )
    def gather_kernel(idx_hbm, table_hbm, out_hbm, idx_v, rows_v, gsems, osems):
        wid = lax.axis_index("s") * _NC + lax.axis_index("c")
        base = wid * per_w
        pltpu.sync_copy(idx_hbm.at[wid], idx_v)

        def start_gather(j, b):
            pltpu.async_copy(table_hbm.at[idx_v.at[j]], rows_v.at[b], gsems[b])

        def wait_gather(b):
            pltpu.make_async_copy(
                table_hbm.at[idx_v.at[0]], rows_v.at[b], gsems[b]
            ).wait()

        def start_out(j, b):
            pltpu.async_copy(
                rows_v.at[b],
                out_hbm.at[pl.ds(base + j * _CHUNK, _CHUNK)],
                osems[b],
            )

        def wait_out(b):
            pltpu.make_async_copy(
                rows_v.at[b], out_hbm.at[pl.ds(base, _CHUNK)], osems[b]
            ).wait()

        for j in range(_LOOKAHEAD):
            start_gather(j, j)

        @pl.loop(0, n_chunk - 1, step=_NBUF)
        def _group(g):
            for b in range(_NBUF):
                j = g + b
                wait_gather(b)
                start_out(j, b)
                s = (b + _LOOKAHEAD) % _NBUF

                @pl.when(j + _LOOKAHEAD < n_chunk)
                def _issue():
                    @pl.when(j >= _NBUF - _LOOKAHEAD)
                    def _recycle():
                        wait_out(s)

                    start_gather(j + _LOOKAHEAD, s)

        tail = n_chunk - 1
        b_tail = tail % _NBUF
        wait_gather(b_tail)
        start_out(tail, b_tail)
        for b in range(_NBUF):
            wait_out(b)

    return gather_kernel


def kernel(token_ids, table):
    b, l = token_ids.shape
    n = b * l
    idx = token_ids.reshape(_NW, n // (_NW * _CHUNK), _CHUNK).astype(jnp.int32)
    out = _make_gather(n)(idx, table)
    return out.reshape(b, l, _D)
